# baseline (speedup 1.0000x reference)
import functools
import math

import jax
import jax.numpy as jnp
import numpy as np
from jax import lax
from jax.experimental import pallas as pl
from jax.experimental.pallas import tpu as pltpu
from jax.experimental.pallas import tpu_sc as plsc

N_PIX = 256
PIXSCALE = 0.025
NV = 64
VEL0 = -3.15
DV = 0.1
SIGMA = 0.8
TRUNCATE = 3.0
FOV_HALF = 0.5 * (N_PIX - 1) * PIXSCALE
HALF = int(math.ceil(TRUNCATE * SIGMA))

M = 2000000
MP = 2097152
ACC = 32 * 128 * 256

_x = np.arange(-HALF, HALF + 1, dtype=np.float32)
_g1 = np.exp(-0.5 * (_x / SIGMA) ** 2)
_g1 = (_g1 / _g1.sum()).astype(np.float32)


def _reflect(j: int) -> int:
    if j < 0:
        return -j
    if j > N_PIX - 1:
        return 2 * (N_PIX - 1) - j
    return j


_B = np.zeros((N_PIX, N_PIX), np.float32)
for _r in range(N_PIX):
    for _d in range(-HALF, HALF + 1):
        _B[_r, _reflect(_r + _d)] += _g1[_d + HALF]
_BE = np.ascontiguousarray(_B[:, 0::2])
_BO = np.ascontiguousarray(_B[:, 1::2])
_BT = np.ascontiguousarray(_B.T)


_PREP_R = 1024
_PREP_GRID = MP // 128 // _PREP_R


def _prep_body(ra_ref, dec_ref, vel_ref, flux_ref, *out_refs):
    pid = pl.program_id(0)
    ra = ra_ref[...]
    dec = dec_ref[...]
    vel = vel_ref[...]
    flux = flux_ref[...]

    x = (ra + FOV_HALF) / PIXSCALE
    y = (dec + FOV_HALF) / PIXSCALE
    v = (vel - VEL0) / DV

    ix0 = jnp.floor(x)
    iy0 = jnp.floor(y)
    iv0 = jnp.floor(v)
    fx = x - ix0
    fy = y - iy0
    fv = v - iv0
    ix0i = ix0.astype(jnp.int32)
    iy0i = iy0.astype(jnp.int32)
    iv0i = iv0.astype(jnp.int32)

    mask = ((ix0i >= 0) & (ix0i < N_PIX - 1)
            & (iy0i >= 0) & (iy0i < N_PIX - 1)
            & (iv0i >= 0) & (iv0i < NV - 1))

    ix0c = jnp.clip(ix0i, 0, N_PIX - 2)
    iy0c = jnp.clip(iy0i, 0, N_PIX - 2)
    iv0c = jnp.clip(iv0i, 0, NV - 2)
    iy1c = iy0c + 1
    iv1c = iv0c + 1

    wx0 = 1.0 - fx
    wx1 = fx
    wy0 = 1.0 - fy
    wy1 = fy
    wv0 = 1.0 - fv
    wv1 = fv

    shp = ra.shape
    slot = (pid * (_PREP_R * 128)
            + lax.broadcasted_iota(jnp.int32, shp, 0) * 128
            + lax.broadcasted_iota(jnp.int32, shp, 1))
    spread = slot & (ACC - 1)

    iv0_even = (iv0c & 1) == 0
    iy0_even = (iy0c & 1) == 0

    o = iter(out_refs)
    for cp in (0, 1):
        want_iv0 = iv0_even if cp == 0 else jnp.logical_not(iv0_even)
        ch = jnp.where(want_iv0, iv0c, iv1c)
        wv = jnp.where(want_iv0, wv0, wv1)
        for rp in (0, 1):
            want_iy0 = iy0_even if rp == 0 else jnp.logical_not(iy0_even)
            row = jnp.where(want_iy0, iy0c, iy1c)
            wy = jnp.where(want_iy0, wy0, wy1)
            base = ((ch >> 1) * 128 + (row >> 1)) * 256 + ix0c
            v0 = jnp.where(mask, flux * ((wx0 * wy) * wv), 0.0)
            v1 = jnp.where(mask, flux * ((wx1 * wy) * wv), 0.0)
            idx0 = jnp.where(mask, base, spread & ~1)
            next(o)[...] = idx0
            next(o)[...] = v0
            next(o)[...] = v1


def _run_prep(ra, dec, vel, flux):
    blk = pl.BlockSpec((_PREP_R, 128), lambda i: (i, 0))
    outs = []
    for _ in range(4):
        outs += [jax.ShapeDtypeStruct((MP // 128, 128), jnp.int32)]
        outs += [jax.ShapeDtypeStruct((MP // 128, 128), jnp.float32)] * 2
    return pl.pallas_call(
        _prep_body,
        grid=(_PREP_GRID,),
        in_specs=[blk] * 4,
        out_specs=[blk] * 12,
        out_shape=outs,
    )(ra, dec, vel, flux)


_CHUNK = 4096
_PER_TILE = MP // 16
_NCHUNK = _PER_TILE // _CHUNK
_NZERO = (ACC // 16) // _CHUNK


def _sc_body(*refs):
    ins = refs[:12]
    out = refs[12]
    (acc, i0_0, i0_1, i1_0, i1_1, v0_0, v0_1, v1_0, v1_1, zbuf,
     ls0, ls1, ss0, ss1) = refs[13:]
    i0 = (i0_0, i0_1)
    i1 = (i1_0, i1_1)
    v0 = (v0_0, v0_1)
    v1 = (v1_0, v1_1)
    lsem = (ls0, ls1)
    ssem = (ss0, ss1)

    c = lax.axis_index("c")
    s = lax.axis_index("s")

    def _zero_zbuf(i, _):
        zbuf[pl.ds(i * 16, 16)] = jnp.zeros((16,), jnp.float32)
        return _

    lax.fori_loop(0, _CHUNK // 16, _zero_zbuf, 0)

    def _zero_acc_slice():
        def body(j, _):
            pltpu.sync_copy(zbuf, acc.at[pl.ds((s * _NZERO + j) * _CHUNK, _CHUNK)])
            return _
        lax.fori_loop(0, _NZERO, body, 0)

    _zero_acc_slice()
    plsc.subcore_barrier()

    for cp in (0, 1):
        @pl.when(c == cp)
        def _process():
            for rp in (0, 1):
                idx0, val0, val1 = ins[3 * (2 * cp + rp): 3 * (2 * cp + rp) + 3]
                base0 = s * _PER_TILE

                def _loads(b, base, go):
                    srcs = (idx0, val0, val1)
                    dsts = (i0[b], v0[b], v1[b])
                    for src, dst in zip(srcs, dsts):
                        d = pltpu.make_async_copy(src.at[pl.ds(base, _CHUNK)],
                                                  dst, lsem[b])
                        if go:
                            d.start()
                        else:
                            d.wait()

                _loads(0, base0, True)
                _loads(1, base0 + _CHUNK, True)

                def jbody(j, carry):
                    for b in (0, 1):
                        k = 2 * j + b
                        base = base0 + k * _CHUNK
                        _loads(b, base, False)

                        def _plus1(t, cc):
                            i1[b][pl.ds(t * 16, 16)] = (
                                i0[b][pl.ds(t * 16, 16)] + 1)
                            return cc
                        lax.fori_loop(0, _CHUNK // 16, _plus1, 0)

                        h0 = pltpu.async_copy(v0[b], acc.at[i0[b]], ssem[b],
                                              add=True)
                        h1 = pltpu.async_copy(v1[b], acc.at[i1[b]], ssem[b],
                                              add=True)
                        h0.wait()
                        h1.wait()

                        @pl.when(k + 2 < _NCHUNK)
                        def _prefetch(b=b, base=base):
                            _loads(b, base + 2 * _CHUNK, True)
                    return carry

                lax.fori_loop(0, _NCHUNK // 2, jbody, 0)
                plsc.subcore_barrier()
                pltpu.sync_copy(acc.at[pl.ds(s * (ACC // 16), ACC // 16)],
                                out.at[cp, rp, pl.ds(s * (ACC // 16), ACC // 16)])
                if rp == 0:
                    _zero_acc_slice()
                    plsc.subcore_barrier()


def _run_scatter(streams):
    mesh = plsc.VectorSubcoreMesh(core_axis_name="c", subcore_axis_name="s")
    kern = pl.kernel(
        _sc_body,
        mesh=mesh,
        out_type=jax.ShapeDtypeStruct((2, 2, ACC), jnp.float32),
        scratch_types=(
            [pltpu.VMEM_SHARED((ACC,), jnp.float32)]
            + [pltpu.VMEM((_CHUNK,), jnp.int32)] * 4
            + [pltpu.VMEM((_CHUNK,), jnp.float32)] * 5
            + [pltpu.SemaphoreType.DMA] * 4
        ),
    )
    return kern(*streams)


def _blur_body(xe_ref, xo_ref, be_ref, bo_ref, bt_ref, out_ref):
    xe = xe_ref[...].reshape(128, 256)
    xo = xo_ref[...].reshape(128, 256)
    y = (jnp.dot(be_ref[...], xe, preferred_element_type=jnp.float32)
         + jnp.dot(bo_ref[...], xo, preferred_element_type=jnp.float32))
    out_ref[...] = jnp.dot(y, bt_ref[...],
                           preferred_element_type=jnp.float32)[None]


def _run_blur(scr):
    xspec = lambda rp: pl.BlockSpec((1, 1, 1, 128, 256),
                                    lambda c, rp=rp: (c % 2, rp, c // 2, 0, 0))
    full = lambda shape: pl.BlockSpec(shape, lambda c: (0,) * len(shape))
    return pl.pallas_call(
        _blur_body,
        grid=(NV,),
        in_specs=[xspec(0), xspec(1),
                  full((N_PIX, 128)), full((N_PIX, 128)), full((N_PIX, N_PIX))],
        out_specs=pl.BlockSpec((1, N_PIX, N_PIX), lambda c: (c, 0, 0)),
        out_shape=jax.ShapeDtypeStruct((NV, N_PIX, N_PIX), jnp.float32),
    )(scr, scr, _BE, _BO, _BT)


def kernel(pos_img, vel_chan, flux):
    pad = MP - M
    ra = jnp.concatenate([pos_img[:, 0], jnp.full((pad,), 1e9, jnp.float32)])
    dec = jnp.concatenate([pos_img[:, 1], jnp.full((pad,), 1e9, jnp.float32)])
    vel = jnp.concatenate([vel_chan, jnp.full((pad,), 1e9, jnp.float32)])
    flx = jnp.concatenate([flux, jnp.zeros((pad,), jnp.float32)])
    shape2 = (MP // 128, 128)
    streams = _run_prep(ra.reshape(shape2), dec.reshape(shape2),
                        vel.reshape(shape2), flx.reshape(shape2))
    flat = [jnp.reshape(a, (MP,)) for a in streams]
    cube_split = _run_scatter(flat)
    scr = cube_split.reshape(2, 2, 32, 128, 256)
    return _run_blur(scr)


# speedup vs baseline: 1.0434x; 1.0434x over previous
import functools
import math

import jax
import jax.numpy as jnp
import numpy as np
from jax import lax
from jax.experimental import pallas as pl
from jax.experimental.pallas import tpu as pltpu
from jax.experimental.pallas import tpu_sc as plsc

N_PIX = 256
PIXSCALE = 0.025
NV = 64
VEL0 = -3.15
DV = 0.1
SIGMA = 0.8
TRUNCATE = 3.0
FOV_HALF = 0.5 * (N_PIX - 1) * PIXSCALE
HALF = int(math.ceil(TRUNCATE * SIGMA))

M = 2000000
MP = 2097152
ACC = 32 * 128 * 256

_x = np.arange(-HALF, HALF + 1, dtype=np.float32)
_g1 = np.exp(-0.5 * (_x / SIGMA) ** 2)
_g1 = (_g1 / _g1.sum()).astype(np.float32)


def _reflect(j: int) -> int:
    if j < 0:
        return -j
    if j > N_PIX - 1:
        return 2 * (N_PIX - 1) - j
    return j


_B = np.zeros((N_PIX, N_PIX), np.float32)
for _r in range(N_PIX):
    for _d in range(-HALF, HALF + 1):
        _B[_r, _reflect(_r + _d)] += _g1[_d + HALF]
_BE = np.ascontiguousarray(_B[:, 0::2])
_BO = np.ascontiguousarray(_B[:, 1::2])
_BT = np.ascontiguousarray(_B.T)


_PREP_R = 1024
_PREP_GRID = MP // 2 // 128 // _PREP_R


def _prep_body(ra_ref, dec_ref, vel_ref, flux_ref, *out_refs):
    pid = pl.program_id(0)
    ra = ra_ref[...]
    dec = dec_ref[...]
    vel = vel_ref[...]
    flux = flux_ref[...]

    x = (ra + FOV_HALF) / PIXSCALE
    y = (dec + FOV_HALF) / PIXSCALE
    v = (vel - VEL0) / DV

    ix0 = jnp.floor(x)
    iy0 = jnp.floor(y)
    iv0 = jnp.floor(v)
    fx = x - ix0
    fy = y - iy0
    fv = v - iv0
    ix0i = ix0.astype(jnp.int32)
    iy0i = iy0.astype(jnp.int32)
    iv0i = iv0.astype(jnp.int32)

    mask = ((ix0i >= 0) & (ix0i < N_PIX - 1)
            & (iy0i >= 0) & (iy0i < N_PIX - 1)
            & (iv0i >= 0) & (iv0i < NV - 1))

    ix0c = jnp.clip(ix0i, 0, N_PIX - 2)
    iy0c = jnp.clip(iy0i, 0, N_PIX - 2)
    iv0c = jnp.clip(iv0i, 0, NV - 2)
    iy1c = iy0c + 1
    iv1c = iv0c + 1

    wx0 = 1.0 - fx
    wx1 = fx
    wy0 = 1.0 - fy
    wy1 = fy
    wv0 = 1.0 - fv
    wv1 = fv

    shp = ra.shape
    slot = (pid * (_PREP_R * 128)
            + lax.broadcasted_iota(jnp.int32, shp, 0) * 128
            + lax.broadcasted_iota(jnp.int32, shp, 1))
    spread = slot & (ACC - 1)

    iv0_even = (iv0c & 1) == 0
    iy0_even = (iy0c & 1) == 0

    o = iter(out_refs)
    for cp in (0, 1):
        want_iv0 = iv0_even if cp == 0 else jnp.logical_not(iv0_even)
        ch = jnp.where(want_iv0, iv0c, iv1c)
        wv = jnp.where(want_iv0, wv0, wv1)
        for rp in (0, 1):
            want_iy0 = iy0_even if rp == 0 else jnp.logical_not(iy0_even)
            row = jnp.where(want_iy0, iy0c, iy1c)
            wy = jnp.where(want_iy0, wy0, wy1)
            base = ((ch >> 1) * 128 + (row >> 1)) * 256 + ix0c
            v0 = jnp.where(mask, flux * ((wx0 * wy) * wv), 0.0)
            v1 = jnp.where(mask, flux * ((wx1 * wy) * wv), 0.0)
            idx0 = jnp.where(mask, base, spread & ~1)
            next(o)[...] = idx0
            next(o)[...] = idx0 + 1
            next(o)[...] = v0
            next(o)[...] = v1


def _run_prep(ra, dec, vel, flux):
    blk = pl.BlockSpec((_PREP_R, 128), lambda i: (i, 0))
    outs = []
    for _ in range(4):
        outs += [jax.ShapeDtypeStruct((_MPH // 128, 128), jnp.int32)] * 2
        outs += [jax.ShapeDtypeStruct((_MPH // 128, 128), jnp.float32)] * 2
    return pl.pallas_call(
        _prep_body,
        grid=(_PREP_GRID,),
        in_specs=[blk] * 4,
        out_specs=[blk] * 16,
        out_shape=outs,
    )(ra, dec, vel, flux)


_CHUNK = 4096
_MPH = MP // 2
_PER_TILE = _MPH // 16
_NCHUNK = _PER_TILE // _CHUNK
_NZERO = (ACC // 16) // _CHUNK


def _sc_body(*refs):
    ins = refs[:16]
    out = refs[16]
    (acc, i0_0, i0_1, i1_0, i1_1, v0_0, v0_1, v1_0, v1_1, zbuf,
     ls0, ls1, ss0, ss1) = refs[17:]
    i0 = (i0_0, i0_1)
    i1 = (i1_0, i1_1)
    v0 = (v0_0, v0_1)
    v1 = (v1_0, v1_1)
    lsem = (ls0, ls1)
    ssem = (ss0, ss1)

    c = lax.axis_index("c")
    s = lax.axis_index("s")

    def _zero_zbuf(i, _):
        zbuf[pl.ds(i * 16, 16)] = jnp.zeros((16,), jnp.float32)
        return _

    lax.fori_loop(0, _CHUNK // 16, _zero_zbuf, 0)

    def _zero_acc_slice():
        def body(j, _):
            pltpu.sync_copy(zbuf, acc.at[pl.ds((s * _NZERO + j) * _CHUNK, _CHUNK)])
            return _
        lax.fori_loop(0, _NZERO, body, 0)

    _zero_acc_slice()
    plsc.subcore_barrier()

    for cp in (0, 1):
        @pl.when(c == cp)
        def _process():
            for rp in (0, 1):
                idx0, idx1, val0, val1 = ins[4 * (2 * cp + rp): 4 * (2 * cp + rp) + 4]
                base0 = s * _PER_TILE

                def _loads(b, base, go):
                    srcs = (idx0, idx1, val0, val1)
                    dsts = (i0[b], i1[b], v0[b], v1[b])
                    for src, dst in zip(srcs, dsts):
                        d = pltpu.make_async_copy(src.at[pl.ds(base, _CHUNK)],
                                                  dst, lsem[b])
                        if go:
                            d.start()
                        else:
                            d.wait()

                _loads(0, base0, True)
                _loads(1, base0 + _CHUNK, True)

                def jbody(j, carry):
                    for b in (0, 1):
                        k = 2 * j + b
                        base = base0 + k * _CHUNK
                        _loads(b, base, False)
                        h0 = pltpu.async_copy(v0[b], acc.at[i0[b]], ssem[b],
                                              add=True)
                        h1 = pltpu.async_copy(v1[b], acc.at[i1[b]], ssem[b],
                                              add=True)
                        h0.wait()
                        h1.wait()

                        @pl.when(k + 2 < _NCHUNK)
                        def _prefetch(b=b, base=base):
                            _loads(b, base + 2 * _CHUNK, True)
                    return carry

                lax.fori_loop(0, _NCHUNK // 2, jbody, 0)
                plsc.subcore_barrier()
                pltpu.sync_copy(acc.at[pl.ds(s * (ACC // 16), ACC // 16)],
                                out.at[cp, rp, pl.ds(s * (ACC // 16), ACC // 16)])
                if rp == 0:
                    _zero_acc_slice()
                    plsc.subcore_barrier()


def _run_scatter(streams):
    mesh = plsc.VectorSubcoreMesh(core_axis_name="c", subcore_axis_name="s")
    kern = pl.kernel(
        _sc_body,
        mesh=mesh,
        out_type=jax.ShapeDtypeStruct((2, 2, ACC), jnp.float32),
        scratch_types=(
            [pltpu.VMEM_SHARED((ACC,), jnp.float32)]
            + [pltpu.VMEM((_CHUNK,), jnp.int32)] * 4
            + [pltpu.VMEM((_CHUNK,), jnp.float32)] * 5
            + [pltpu.SemaphoreType.DMA] * 4
        ),
    )
    return kern(*streams)


def _blur_body(xe0_ref, xo0_ref, xe1_ref, xo1_ref, be_ref, bo_ref, bt_ref,
               out_ref):
    xe = (xe0_ref[...] + xe1_ref[...]).reshape(128, 256)
    xo = (xo0_ref[...] + xo1_ref[...]).reshape(128, 256)
    y = (jnp.dot(be_ref[...], xe, preferred_element_type=jnp.float32)
         + jnp.dot(bo_ref[...], xo, preferred_element_type=jnp.float32))
    out_ref[...] = jnp.dot(y, bt_ref[...],
                           preferred_element_type=jnp.float32)[None]


def _run_blur(scr0, scr1):
    xspec = lambda rp: pl.BlockSpec((1, 1, 1, 128, 256),
                                    lambda c, rp=rp: (c % 2, rp, c // 2, 0, 0))
    full = lambda shape: pl.BlockSpec(shape, lambda c: (0,) * len(shape))
    return pl.pallas_call(
        _blur_body,
        grid=(NV,),
        in_specs=[xspec(0), xspec(1), xspec(0), xspec(1),
                  full((N_PIX, 128)), full((N_PIX, 128)), full((N_PIX, N_PIX))],
        out_specs=pl.BlockSpec((1, N_PIX, N_PIX), lambda c: (c, 0, 0)),
        out_shape=jax.ShapeDtypeStruct((NV, N_PIX, N_PIX), jnp.float32),
    )(scr0, scr0, scr1, scr1, _BE, _BO, _BT)


def kernel(pos_img, vel_chan, flux):
    pad = MP - M
    ra = jnp.concatenate([pos_img[:, 0], jnp.full((pad,), 1e9, jnp.float32)])
    dec = jnp.concatenate([pos_img[:, 1], jnp.full((pad,), 1e9, jnp.float32)])
    vel = jnp.concatenate([vel_chan, jnp.full((pad,), 1e9, jnp.float32)])
    flx = jnp.concatenate([flux, jnp.zeros((pad,), jnp.float32)])
    shape3 = (2, _MPH // 128, 128)
    halves = [a.reshape(shape3) for a in (ra, dec, vel, flx)]
    scrs = []
    for h in (0, 1):
        streams = _run_prep(*(a[h] for a in halves))
        flat = [jnp.reshape(a, (_MPH,)) for a in streams]
        scrs.append(_run_scatter(flat).reshape(2, 2, 32, 128, 256))
    return _run_blur(*scrs)


# speedup vs baseline: 1.0843x; 1.0392x over previous
import functools
import math

import jax
import jax.numpy as jnp
import numpy as np
from jax import lax
from jax.experimental import pallas as pl
from jax.experimental.pallas import tpu as pltpu
from jax.experimental.pallas import tpu_sc as plsc

N_PIX = 256
PIXSCALE = 0.025
NV = 64
VEL0 = -3.15
DV = 0.1
SIGMA = 0.8
TRUNCATE = 3.0
FOV_HALF = 0.5 * (N_PIX - 1) * PIXSCALE
HALF = int(math.ceil(TRUNCATE * SIGMA))

M = 2000000
MP = 2097152
ACC = 32 * 128 * 256

_x = np.arange(-HALF, HALF + 1, dtype=np.float32)
_g1 = np.exp(-0.5 * (_x / SIGMA) ** 2)
_g1 = (_g1 / _g1.sum()).astype(np.float32)


def _reflect(j: int) -> int:
    if j < 0:
        return -j
    if j > N_PIX - 1:
        return 2 * (N_PIX - 1) - j
    return j


_B = np.zeros((N_PIX, N_PIX), np.float32)
for _r in range(N_PIX):
    for _d in range(-HALF, HALF + 1):
        _B[_r, _reflect(_r + _d)] += _g1[_d + HALF]
_BE = np.ascontiguousarray(_B[:, 0::2])
_BO = np.ascontiguousarray(_B[:, 1::2])
_BT = np.ascontiguousarray(_B.T)


_PREP_R = 1024
_PREP_GRID = MP // 128 // _PREP_R


def _prep_body(ra_ref, dec_ref, vel_ref, flux_ref, *out_refs):
    pid = pl.program_id(0)
    ra = ra_ref[...]
    dec = dec_ref[...]
    vel = vel_ref[...]
    flux = flux_ref[...]

    x = (ra + FOV_HALF) / PIXSCALE
    y = (dec + FOV_HALF) / PIXSCALE
    v = (vel - VEL0) / DV

    ix0 = jnp.floor(x)
    iy0 = jnp.floor(y)
    iv0 = jnp.floor(v)
    fx = x - ix0
    fy = y - iy0
    fv = v - iv0
    ix0i = ix0.astype(jnp.int32)
    iy0i = iy0.astype(jnp.int32)
    iv0i = iv0.astype(jnp.int32)

    mask = ((ix0i >= 0) & (ix0i < N_PIX - 1)
            & (iy0i >= 0) & (iy0i < N_PIX - 1)
            & (iv0i >= 0) & (iv0i < NV - 1))

    ix0c = jnp.clip(ix0i, 0, N_PIX - 2)
    iy0c = jnp.clip(iy0i, 0, N_PIX - 2)
    iv0c = jnp.clip(iv0i, 0, NV - 2)
    iy1c = iy0c + 1
    iv1c = iv0c + 1

    wx0 = 1.0 - fx
    wx1 = fx
    wy0 = 1.0 - fy
    wy1 = fy
    wv0 = 1.0 - fv
    wv1 = fv

    shp = ra.shape
    slot = (pid * (_PREP_R * 128)
            + lax.broadcasted_iota(jnp.int32, shp, 0) * 128
            + lax.broadcasted_iota(jnp.int32, shp, 1))
    spread = slot & (ACC - 1)

    iv0_even = (iv0c & 1) == 0
    iy0_even = (iy0c & 1) == 0

    o = iter(out_refs)
    for cp in (0, 1):
        want_iv0 = iv0_even if cp == 0 else jnp.logical_not(iv0_even)
        ch = jnp.where(want_iv0, iv0c, iv1c)
        wv = jnp.where(want_iv0, wv0, wv1)
        for rp in (0, 1):
            want_iy0 = iy0_even if rp == 0 else jnp.logical_not(iy0_even)
            row = jnp.where(want_iy0, iy0c, iy1c)
            wy = jnp.where(want_iy0, wy0, wy1)
            base = ((ch >> 1) * 128 + (row >> 1)) * 256 + ix0c
            v0 = jnp.where(mask, flux * ((wx0 * wy) * wv), 0.0)
            v1 = jnp.where(mask, flux * ((wx1 * wy) * wv), 0.0)
            idx0 = jnp.where(mask, base, spread & ~1)
            next(o)[...] = idx0
            next(o)[...] = idx0 + 1
            next(o)[...] = v0
            next(o)[...] = v1


def _run_prep(ra, dec, vel, flux):
    blk = pl.BlockSpec((_PREP_R, 128), lambda i: (i, 0))
    outs = []
    for _ in range(4):
        outs += [jax.ShapeDtypeStruct((_MPH // 128, 128), jnp.int32)] * 2
        outs += [jax.ShapeDtypeStruct((_MPH // 128, 128), jnp.float32)] * 2
    return pl.pallas_call(
        _prep_body,
        grid=(_PREP_GRID,),
        in_specs=[blk] * 4,
        out_specs=[blk] * 16,
        out_shape=outs,
    )(ra, dec, vel, flux)


_CHUNK = 4096
_MPH = MP
_PER_TILE = _MPH // 16
_NCHUNK = _PER_TILE // _CHUNK
_NZERO = (ACC // 16) // _CHUNK


def _sc_body(*refs):
    ins = refs[:16]
    out = refs[16]
    (acc, i0_0, i0_1, i0_2, i1_0, i1_1, i1_2, v0_0, v0_1, v0_2,
     v1_0, v1_1, v1_2, zbuf, ls0, ls1, ls2, ss0, ss1, ss2) = refs[17:]
    i0 = (i0_0, i0_1, i0_2)
    i1 = (i1_0, i1_1, i1_2)
    v0 = (v0_0, v0_1, v0_2)
    v1 = (v1_0, v1_1, v1_2)
    lsem = (ls0, ls1, ls2)
    ssem = (ss0, ss1, ss2)

    c = lax.axis_index("c")
    s = lax.axis_index("s")

    def _zero_zbuf(i, _):
        zbuf[pl.ds(i * 16, 16)] = jnp.zeros((16,), jnp.float32)
        return _

    lax.fori_loop(0, _CHUNK // 16, _zero_zbuf, 0)

    def _zero_acc_slice():
        def body(j, _):
            pltpu.sync_copy(zbuf, acc.at[pl.ds((s * _NZERO + j) * _CHUNK, _CHUNK)])
            return _
        lax.fori_loop(0, _NZERO, body, 0)

    _zero_acc_slice()
    plsc.subcore_barrier()

    for cp in (0, 1):
        @pl.when(c == cp)
        def _process():
            for rp in (0, 1):
                idx0, idx1, val0, val1 = ins[4 * (2 * cp + rp): 4 * (2 * cp + rp) + 4]
                base0 = s * _PER_TILE

                def _loads(b, base, go):
                    srcs = (idx0, idx1, val0, val1)
                    dsts = (i0[b], i1[b], v0[b], v1[b])
                    for src, dst in zip(srcs, dsts):
                        d = pltpu.make_async_copy(src.at[pl.ds(base, _CHUNK)],
                                                  dst, lsem[b])
                        if go:
                            d.start()
                        else:
                            d.wait()

                def _drain_scatter(b):
                    pltpu.make_async_copy(v0[b], acc.at[i0[b]], ssem[b]).wait()
                    pltpu.make_async_copy(v1[b], acc.at[i1[b]], ssem[b]).wait()

                def _chunk_step(b, k):
                    bn = (b + 1) % 3

                    @pl.when(k >= 2)
                    def _settle():
                        _drain_scatter(bn)

                    @pl.when(k + 1 < _NCHUNK)
                    def _prefetch():
                        _loads(bn, base0 + (k + 1) * _CHUNK, True)

                    _loads(b, base0 + k * _CHUNK, False)
                    pltpu.async_copy(v0[b], acc.at[i0[b]], ssem[b], add=True)
                    pltpu.async_copy(v1[b], acc.at[i1[b]], ssem[b], add=True)

                _loads(0, base0, True)

                def jbody(j, carry):
                    for b in (0, 1, 2):
                        _chunk_step(b, 3 * j + b)
                    return carry

                _ntrip = _NCHUNK // 3
                lax.fori_loop(0, _ntrip, jbody, 0)
                for t in range(_NCHUNK - 3 * _ntrip):
                    _chunk_step(t, 3 * _ntrip + t)
                _drain_scatter((_NCHUNK - 2) % 3)
                _drain_scatter((_NCHUNK - 1) % 3)
                plsc.subcore_barrier()
                pltpu.sync_copy(acc.at[pl.ds(s * (ACC // 16), ACC // 16)],
                                out.at[cp, rp, pl.ds(s * (ACC // 16), ACC // 16)])
                if rp == 0:
                    _zero_acc_slice()
                    plsc.subcore_barrier()


def _run_scatter(streams):
    mesh = plsc.VectorSubcoreMesh(core_axis_name="c", subcore_axis_name="s")
    kern = pl.kernel(
        _sc_body,
        mesh=mesh,
        out_type=jax.ShapeDtypeStruct((2, 2, ACC), jnp.float32),
        scratch_types=(
            [pltpu.VMEM_SHARED((ACC,), jnp.float32)]
            + [pltpu.VMEM((_CHUNK,), jnp.int32)] * 6
            + [pltpu.VMEM((_CHUNK,), jnp.float32)] * 7
            + [pltpu.SemaphoreType.DMA] * 6
        ),
    )
    return kern(*streams)


def _blur_body(xe_ref, xo_ref, be_ref, bo_ref, bt_ref, out_ref):
    xe = xe_ref[...].reshape(128, 256)
    xo = xo_ref[...].reshape(128, 256)
    y = (jnp.dot(be_ref[...], xe, preferred_element_type=jnp.float32)
         + jnp.dot(bo_ref[...], xo, preferred_element_type=jnp.float32))
    out_ref[...] = jnp.dot(y, bt_ref[...],
                           preferred_element_type=jnp.float32)[None]


def _run_blur(scr):
    xspec = lambda rp: pl.BlockSpec((1, 1, 1, 128, 256),
                                    lambda c, rp=rp: (c % 2, rp, c // 2, 0, 0))
    full = lambda shape: pl.BlockSpec(shape, lambda c: (0,) * len(shape))
    return pl.pallas_call(
        _blur_body,
        grid=(NV,),
        in_specs=[xspec(0), xspec(1),
                  full((N_PIX, 128)), full((N_PIX, 128)), full((N_PIX, N_PIX))],
        out_specs=pl.BlockSpec((1, N_PIX, N_PIX), lambda c: (c, 0, 0)),
        out_shape=jax.ShapeDtypeStruct((NV, N_PIX, N_PIX), jnp.float32),
    )(scr, scr, _BE, _BO, _BT)


def kernel(pos_img, vel_chan, flux):
    pad = MP - M
    ra = jnp.concatenate([pos_img[:, 0], jnp.full((pad,), 1e9, jnp.float32)])
    dec = jnp.concatenate([pos_img[:, 1], jnp.full((pad,), 1e9, jnp.float32)])
    vel = jnp.concatenate([vel_chan, jnp.full((pad,), 1e9, jnp.float32)])
    flx = jnp.concatenate([flux, jnp.zeros((pad,), jnp.float32)])
    shape2 = (MP // 128, 128)
    streams = _run_prep(ra.reshape(shape2), dec.reshape(shape2),
                        vel.reshape(shape2), flx.reshape(shape2))
    flat = [jnp.reshape(a, (MP,)) for a in streams]
    scr = _run_scatter(flat).reshape(2, 2, 32, 128, 256)
    return _run_blur(scr)


# speedup vs baseline: 1.1073x; 1.0212x over previous
import functools
import math

import jax
import jax.numpy as jnp
import numpy as np
from jax import lax
from jax.experimental import pallas as pl
from jax.experimental.pallas import tpu as pltpu
from jax.experimental.pallas import tpu_sc as plsc

N_PIX = 256
PIXSCALE = 0.025
NV = 64
VEL0 = -3.15
DV = 0.1
SIGMA = 0.8
TRUNCATE = 3.0
FOV_HALF = 0.5 * (N_PIX - 1) * PIXSCALE
HALF = int(math.ceil(TRUNCATE * SIGMA))

M = 2000000
MP = 2097152
ACC = 32 * 128 * 256

_x = np.arange(-HALF, HALF + 1, dtype=np.float32)
_g1 = np.exp(-0.5 * (_x / SIGMA) ** 2)
_g1 = (_g1 / _g1.sum()).astype(np.float32)


def _reflect(j: int) -> int:
    if j < 0:
        return -j
    if j > N_PIX - 1:
        return 2 * (N_PIX - 1) - j
    return j


_B = np.zeros((N_PIX, N_PIX), np.float32)
for _r in range(N_PIX):
    for _d in range(-HALF, HALF + 1):
        _B[_r, _reflect(_r + _d)] += _g1[_d + HALF]
_BE = np.ascontiguousarray(_B[:, 0::2])
_BO = np.ascontiguousarray(_B[:, 1::2])
_BT = np.ascontiguousarray(_B.T)


_PREP_R = 1024
_PREP_GRID = MP // 128 // _PREP_R


def _prep_body(ra_ref, dec_ref, vel_ref, flux_ref, *out_refs):
    pid = pl.program_id(0)
    ra = ra_ref[...]
    dec = dec_ref[...]
    vel = vel_ref[...]
    flux = flux_ref[...]

    x = (ra + FOV_HALF) / PIXSCALE
    y = (dec + FOV_HALF) / PIXSCALE
    v = (vel - VEL0) / DV

    ix0 = jnp.floor(x)
    iy0 = jnp.floor(y)
    iv0 = jnp.floor(v)
    fx = x - ix0
    fy = y - iy0
    fv = v - iv0
    ix0i = ix0.astype(jnp.int32)
    iy0i = iy0.astype(jnp.int32)
    iv0i = iv0.astype(jnp.int32)

    shp = ra.shape
    slot = (pid * (_PREP_R * 128)
            + lax.broadcasted_iota(jnp.int32, shp, 0) * 128
            + lax.broadcasted_iota(jnp.int32, shp, 1))

    mask = ((ix0i >= 0) & (ix0i < N_PIX - 1)
            & (iy0i >= 0) & (iy0i < N_PIX - 1)
            & (iv0i >= 0) & (iv0i < NV - 1)
            & (slot < M))

    ix0c = jnp.clip(ix0i, 0, N_PIX - 2)
    iy0c = jnp.clip(iy0i, 0, N_PIX - 2)
    iv0c = jnp.clip(iv0i, 0, NV - 2)
    iy1c = iy0c + 1
    iv1c = iv0c + 1

    wx0 = 1.0 - fx
    wx1 = fx
    wy0 = 1.0 - fy
    wy1 = fy
    wv0 = 1.0 - fv
    wv1 = fv

    spread = slot & (ACC - 1)

    iv0_even = (iv0c & 1) == 0
    iy0_even = (iy0c & 1) == 0

    o = iter(out_refs)
    for cp in (0, 1):
        want_iv0 = iv0_even if cp == 0 else jnp.logical_not(iv0_even)
        ch = jnp.where(want_iv0, iv0c, iv1c)
        wv = jnp.where(want_iv0, wv0, wv1)
        for rp in (0, 1):
            want_iy0 = iy0_even if rp == 0 else jnp.logical_not(iy0_even)
            row = jnp.where(want_iy0, iy0c, iy1c)
            wy = jnp.where(want_iy0, wy0, wy1)
            base = ((ch >> 1) * 128 + (row >> 1)) * 256 + ix0c
            v0 = jnp.where(mask, flux * ((wx0 * wy) * wv), 0.0)
            v1 = jnp.where(mask, flux * ((wx1 * wy) * wv), 0.0)
            idx0 = jnp.where(mask, base, spread & ~1)
            next(o)[...] = idx0
            next(o)[...] = idx0 + 1
            next(o)[...] = v0
            next(o)[...] = v1


def _run_prep(ra, dec, vel, flux):
    blk = pl.BlockSpec((_PREP_R, 128), lambda i: (i, 0))
    outs = []
    for _ in range(4):
        outs += [jax.ShapeDtypeStruct((_MPH // 128, 128), jnp.int32)] * 2
        outs += [jax.ShapeDtypeStruct((_MPH // 128, 128), jnp.float32)] * 2
    return pl.pallas_call(
        _prep_body,
        grid=(_PREP_GRID,),
        in_specs=[blk] * 4,
        out_specs=[blk] * 16,
        out_shape=outs,
    )(ra, dec, vel, flux)


_CHUNK = 4096
_MPH = MP
_PER_TILE = _MPH // 16
_NCHUNK = _PER_TILE // _CHUNK
_NZERO = (ACC // 16) // _CHUNK


def _sc_body(*refs):
    ins = refs[:16]
    out = refs[16]
    (acc, i0_0, i0_1, i0_2, i1_0, i1_1, i1_2, v0_0, v0_1, v0_2,
     v1_0, v1_1, v1_2, zbuf, ls0, ls1, ls2, ss0, ss1, ss2) = refs[17:]
    i0 = (i0_0, i0_1, i0_2)
    i1 = (i1_0, i1_1, i1_2)
    v0 = (v0_0, v0_1, v0_2)
    v1 = (v1_0, v1_1, v1_2)
    lsem = (ls0, ls1, ls2)
    ssem = (ss0, ss1, ss2)

    c = lax.axis_index("c")
    s = lax.axis_index("s")

    def _zero_zbuf(i, _):
        zbuf[pl.ds(i * 16, 16)] = jnp.zeros((16,), jnp.float32)
        return _

    lax.fori_loop(0, _CHUNK // 16, _zero_zbuf, 0)

    def _zero_acc_slice():
        def body(j, _):
            pltpu.sync_copy(zbuf, acc.at[pl.ds((s * _NZERO + j) * _CHUNK, _CHUNK)])
            return _
        lax.fori_loop(0, _NZERO, body, 0)

    _zero_acc_slice()
    plsc.subcore_barrier()

    for cp in (0, 1):
        @pl.when(c == cp)
        def _process():
            for rp in (0, 1):
                idx0, idx1, val0, val1 = ins[4 * (2 * cp + rp): 4 * (2 * cp + rp) + 4]
                base0 = s * _PER_TILE

                def _loads(b, base, go):
                    srcs = (idx0, idx1, val0, val1)
                    dsts = (i0[b], i1[b], v0[b], v1[b])
                    for src, dst in zip(srcs, dsts):
                        d = pltpu.make_async_copy(src.at[pl.ds(base, _CHUNK)],
                                                  dst, lsem[b])
                        if go:
                            d.start()
                        else:
                            d.wait()

                def _drain_scatter(b):
                    pltpu.make_async_copy(v0[b], acc.at[i0[b]], ssem[b]).wait()
                    pltpu.make_async_copy(v1[b], acc.at[i1[b]], ssem[b]).wait()

                def _chunk_step(b, k):
                    bn = (b + 1) % 3

                    @pl.when(k >= 2)
                    def _settle():
                        _drain_scatter(bn)

                    @pl.when(k + 1 < _NCHUNK)
                    def _prefetch():
                        _loads(bn, base0 + (k + 1) * _CHUNK, True)

                    _loads(b, base0 + k * _CHUNK, False)
                    pltpu.async_copy(v0[b], acc.at[i0[b]], ssem[b], add=True)
                    pltpu.async_copy(v1[b], acc.at[i1[b]], ssem[b], add=True)

                _loads(0, base0, True)

                def jbody(j, carry):
                    for b in (0, 1, 2):
                        _chunk_step(b, 3 * j + b)
                    return carry

                _ntrip = _NCHUNK // 3
                lax.fori_loop(0, _ntrip, jbody, 0)
                for t in range(_NCHUNK - 3 * _ntrip):
                    _chunk_step(t, 3 * _ntrip + t)
                _drain_scatter((_NCHUNK - 2) % 3)
                _drain_scatter((_NCHUNK - 1) % 3)
                plsc.subcore_barrier()
                pltpu.sync_copy(acc.at[pl.ds(s * (ACC // 16), ACC // 16)],
                                out.at[cp, rp, pl.ds(s * (ACC // 16), ACC // 16)])
                if rp == 0:
                    _zero_acc_slice()
                    plsc.subcore_barrier()


def _run_scatter(streams):
    mesh = plsc.VectorSubcoreMesh(core_axis_name="c", subcore_axis_name="s")
    kern = pl.kernel(
        _sc_body,
        mesh=mesh,
        out_type=jax.ShapeDtypeStruct((2, 2, ACC), jnp.float32),
        scratch_types=(
            [pltpu.VMEM_SHARED((ACC,), jnp.float32)]
            + [pltpu.VMEM((_CHUNK,), jnp.int32)] * 6
            + [pltpu.VMEM((_CHUNK,), jnp.float32)] * 7
            + [pltpu.SemaphoreType.DMA] * 6
        ),
    )
    return kern(*streams)


def _blur_body(xe_ref, xo_ref, be_ref, bo_ref, bt_ref, out_ref):
    xe = xe_ref[...].reshape(128, 256)
    xo = xo_ref[...].reshape(128, 256)
    y = (jnp.dot(be_ref[...], xe, preferred_element_type=jnp.float32)
         + jnp.dot(bo_ref[...], xo, preferred_element_type=jnp.float32))
    out_ref[...] = jnp.dot(y, bt_ref[...],
                           preferred_element_type=jnp.float32)[None]


def _run_blur(scr):
    xspec = lambda rp: pl.BlockSpec((1, 1, 1, 128, 256),
                                    lambda c, rp=rp: (c % 2, rp, c // 2, 0, 0))
    full = lambda shape: pl.BlockSpec(shape, lambda c: (0,) * len(shape))
    return pl.pallas_call(
        _blur_body,
        grid=(NV,),
        in_specs=[xspec(0), xspec(1),
                  full((N_PIX, 128)), full((N_PIX, 128)), full((N_PIX, N_PIX))],
        out_specs=pl.BlockSpec((1, N_PIX, N_PIX), lambda c: (c, 0, 0)),
        out_shape=jax.ShapeDtypeStruct((NV, N_PIX, N_PIX), jnp.float32),
    )(scr, scr, _BE, _BO, _BT)


def kernel(pos_img, vel_chan, flux):
    shape2 = (M // 128, 128)
    streams = _run_prep(pos_img[:, 0].reshape(shape2),
                        pos_img[:, 1].reshape(shape2),
                        vel_chan.reshape(shape2), flux.reshape(shape2))
    flat = [jnp.reshape(a, (MP,)) for a in streams]
    scr = _run_scatter(flat).reshape(2, 2, 32, 128, 256)
    return _run_blur(scr)
